# Initial kernel scaffold; baseline (speedup 1.0000x reference)
import jax
import jax.numpy as jnp
from jax.experimental import pallas as pl


def kernel(x, p, sid_euc, tid_euc, W1a, b1a, g1, be1, W1b, b1b, Wfc, bfc, g2a, be2a, W2a, b2a, g2b, be2b, W2b, b2b):
    raise NotImplementedError("write your pallas kernel here")


# speedup vs baseline: 5.5123x; 5.5123x over previous
import functools

import jax
import jax.numpy as jnp
from jax import lax
from jax.experimental import pallas as pl
from jax.experimental.pallas import tpu as pltpu
from jax.experimental.pallas import tpu_sc as plsc

B, N, K, C, S = 2, 2048, 16, 256, 8
CS = C // S
P = B * N
E = B * N * K
TW = 2 * C
NC, NS = 2, 16
NW = NC * NS
EPW = E // NW
G = 64
NCHUNK = EPW // G

_dot = functools.partial(
    lax.dot_general,
    dimension_numbers=(((1,), (0,)), ((), ())),
    preferred_element_type=jnp.float32,
)


def _tables_body(x_ref, p_ref, w1a_ref, b1a_ref, s1_ref, be1_ref, w1b_ref,
                 b1b_ref, wfc_ref, bfc_ref, ta_ref, ts_ref):
    h = _dot(p_ref[...], w1a_ref[...]) + b1a_ref[...]
    h = jnp.maximum(h * s1_ref[...] + be1_ref[...], 0.0)
    pf = _dot(h, w1b_ref[...]) + b1b_ref[...]
    qkv = _dot(x_ref[...], wfc_ref[...]) + bfc_ref[...]
    q = qkv[:, :C]
    k = qkv[:, C:2 * C]
    v = qkv[:, 2 * C:]
    ta_ref[...] = jnp.concatenate([q + pf, pf], axis=1)
    ts_ref[...] = jnp.concatenate([-(k + pf), v - pf], axis=1)


def _build_tables(x2, p2, w1a, b1a, s1, be1, w1b, b1b, wfc, bfc):
    rows = 512
    grid = (P // rows,)
    full = lambda shape: pl.BlockSpec(shape, lambda i: (0, 0))
    return pl.pallas_call(
        _tables_body,
        grid=grid,
        in_specs=[
            pl.BlockSpec((rows, C), lambda i: (i, 0)),
            pl.BlockSpec((rows, 8), lambda i: (i, 0)),
            full((8, 8)), full((1, 8)), full((1, 8)), full((1, 8)),
            full((8, C)), full((1, C)),
            full((C, 3 * C)), full((1, 3 * C)),
        ],
        out_specs=[
            pl.BlockSpec((rows, TW), lambda i: (i, 0)),
            pl.BlockSpec((rows, TW), lambda i: (i, 0)),
        ],
        out_shape=[
            jax.ShapeDtypeStruct((P, TW), jnp.float32),
            jax.ShapeDtypeStruct((P, TW), jnp.float32),
        ],
    )(x2, p2, w1a, b1a, s1, be1, w1b, b1b, wfc, bfc)


def _sc_gather_body(ta_hbm, ts_hbm, tid_hbm, sid_hbm, ga_hbm, gs_hbm,
                    tid_v, sid_v, buf_a, buf_b, sem_a, sem_b):
    wid = lax.axis_index("s") * NC + lax.axis_index("c")
    base = wid * EPW
    pltpu.sync_copy(tid_hbm.at[pl.ds(base, EPW)], tid_v)
    pltpu.sync_copy(sid_hbm.at[pl.ds(base, EPW)], sid_v)

    def body(g, carry):
        off = base + g * G
        cp_a = pltpu.async_copy(
            ta_hbm.at[tid_v.at[pl.ds(g * G, G)]], buf_a, sem_a)
        cp_b = pltpu.async_copy(
            ts_hbm.at[sid_v.at[pl.ds(g * G, G)]], buf_b, sem_b)
        cp_a.wait()
        pltpu.sync_copy(buf_a, ga_hbm.at[pl.ds(off, G)])
        cp_b.wait()
        pltpu.sync_copy(buf_b, gs_hbm.at[pl.ds(off, G)])
        return carry

    lax.fori_loop(0, NCHUNK, body, 0)


def _sc_gather(ta, ts, tid, sid):
    mesh = plsc.VectorSubcoreMesh(
        core_axis_name="c", subcore_axis_name="s",
        num_cores=NC, num_subcores=NS)
    fn = pl.kernel(
        _sc_gather_body,
        out_type=[
            jax.ShapeDtypeStruct((E, TW), jnp.float32),
            jax.ShapeDtypeStruct((E, TW), jnp.float32),
        ],
        mesh=mesh,
        scratch_types=[
            pltpu.VMEM((EPW,), jnp.int32),
            pltpu.VMEM((EPW,), jnp.int32),
            pltpu.VMEM((G, TW), jnp.float32),
            pltpu.VMEM((G, TW), jnp.float32),
            pltpu.SemaphoreType.DMA,
            pltpu.SemaphoreType.DMA,
        ],
    )
    return fn(ta, ts, tid, sid)


def _mlp_body(ga_ref, gs_ref, sa_ref, bea_ref, w2a_ref, b2a_ref, sb_ref,
              beb_ref, w2b_ref, b2b_ref, emat_ref, out_ref):
    rows = out_ref.shape[0]
    w = ga_ref[...] + gs_ref[...]
    d = w[:, :C]
    v2 = w[:, C:]
    t = jnp.maximum(d * sa_ref[...] + bea_ref[...], 0.0)
    l1 = _dot(t, w2a_ref[...]) + b2a_ref[...]
    t2 = jnp.maximum(l1 * sb_ref[...] + beb_ref[...], 0.0)
    lg = _dot(t2, w2b_ref[...]) + b2b_ref[...]
    lg3 = lg.reshape(rows, K, CS)
    m = jnp.max(lg3, axis=1, keepdims=True)
    ex = jnp.exp(lg3 - m)
    den = jnp.sum(ex, axis=1, keepdims=True)
    a2 = (ex / den).reshape(rows * K, CS)
    exp_a = _dot(a2, emat_ref[...])
    out_ref[...] = jnp.sum((exp_a * v2).reshape(rows, K, C), axis=1)


def _mlp_reduce(ga, gs, sa, bea, w2a, b2a, sb, beb, w2b, b2b, emat):
    rows = 64
    eb = rows * K
    grid = (P // rows,)
    full = lambda shape: pl.BlockSpec(shape, lambda i: (0, 0))
    return pl.pallas_call(
        _mlp_body,
        grid=grid,
        in_specs=[
            pl.BlockSpec((eb, TW), lambda i: (i, 0)),
            pl.BlockSpec((eb, TW), lambda i: (i, 0)),
            full((1, C)), full((1, C)),
            full((C, CS)), full((1, CS)),
            full((1, CS)), full((1, CS)),
            full((CS, CS)), full((1, CS)),
            full((CS, C)),
        ],
        out_specs=pl.BlockSpec((rows, C), lambda i: (i, 0)),
        out_shape=jax.ShapeDtypeStruct((P, C), jnp.float32),
    )(ga, gs, sa, bea, w2a, b2a, sb, beb, w2b, b2b, emat)


def kernel(x, p, sid_euc, tid_euc, W1a, b1a, g1, be1, W1b, b1b, Wfc, bfc,
           g2a, be2a, W2a, b2a, g2b, be2b, W2b, b2b):
    inv = 1.0 / jnp.sqrt(jnp.float32(1.0 + 1e-5))
    x2 = x.reshape(P, C)
    p2 = jnp.pad(p.reshape(P, 3), ((0, 0), (0, 5)))
    w1a = jnp.pad(W1a, ((0, 5), (0, 5)))
    w1b = jnp.pad(W1b, ((0, 5), (0, 0)))
    b1a8 = jnp.pad(b1a, (0, 5)).reshape(1, 8)
    s1 = (jnp.pad(g1, (0, 5)) * inv).reshape(1, 8)
    be18 = jnp.pad(be1, (0, 5)).reshape(1, 8)

    ta, ts = _build_tables(
        x2, p2, w1a, b1a8, s1, be18, w1b, b1b.reshape(1, C),
        Wfc, bfc.reshape(1, 3 * C))

    tid = tid_euc.reshape(E).astype(jnp.int32)
    sid = sid_euc.reshape(E).astype(jnp.int32)
    ga, gs = _sc_gather(ta, ts, tid, sid)

    emat = jnp.repeat(jnp.eye(CS, dtype=jnp.float32), S, axis=1)
    out = _mlp_reduce(
        ga, gs,
        (g2a * inv).reshape(1, C), be2a.reshape(1, C),
        W2a, b2a.reshape(1, CS),
        (g2b * inv).reshape(1, CS), be2b.reshape(1, CS),
        W2b, b2b.reshape(1, CS), emat)
    return out.reshape(B, N, C)
